# Initial kernel scaffold; baseline (speedup 1.0000x reference)
import jax
import jax.numpy as jnp
from jax.experimental import pallas as pl


def kernel(x, W1, b1, W2, b2):
    raise NotImplementedError("write your pallas kernel here")


# speedup vs baseline: 28.2884x; 28.2884x over previous
import functools

import jax
import jax.numpy as jnp
import numpy as np
from jax.experimental import pallas as pl
from jax.experimental.pallas import tpu as pltpu

Z = 128
N1 = 131072
RW = 1024
C2 = 4096
KW1 = 26214
KW2 = 819

_I32_MIN = np.int32(-2147483648)
_I32_MAX = np.int32(2147483647)


def _mono(x):
    b = jax.lax.bitcast_convert_type(x, jnp.int32)
    return b ^ ((b >> 31) & jnp.int32(0x7FFFFFFF))


def _avg_floor(lo, hi):
    return (lo >> 1) + (hi >> 1) + (lo & hi & 1)


def _mv_kernel(x_ref, w_ref, b_ref, o_ref):
    acc = jax.lax.dot_general(
        x_ref[...], w_ref[...],
        dimension_numbers=(((1,), (1,)), ((), ())),
        preferred_element_type=jnp.float32)
    o_ref[...] = acc + b_ref[...]


def _fc1(x2, W1, b1w):
    BLK = 8192
    grid = N1 // BLK
    return pl.pallas_call(
        _mv_kernel,
        grid=(grid,),
        in_specs=[
            pl.BlockSpec((1, Z), lambda i: (0, 0)),
            pl.BlockSpec((BLK, Z), lambda i: (i, 0)),
            pl.BlockSpec((1, BLK), lambda i: (0, i)),
        ],
        out_specs=pl.BlockSpec((1, BLK), lambda i: (0, i)),
        out_shape=jax.ShapeDtypeStruct((1, N1), jnp.float32),
    )(x2, W1, b1w)


def _kw1_kernel(h_ref, o_ref):
    h = h_ref[...]
    m = _mono(h)

    def body(_, c):
        lo, hi, chi = c
        mid = _avg_floor(lo, hi)
        cnt = jnp.sum((m > mid).astype(jnp.int32))
        ge = cnt >= KW1
        return (jnp.where(ge, mid, lo),
                jnp.where(ge, hi, mid),
                jnp.where(ge, chi, cnt))

    lo, hi, chi = jax.lax.fori_loop(
        0, 32, body, (jnp.int32(_I32_MIN), jnp.int32(_I32_MAX), jnp.int32(0)))
    s = hi
    need = KW1 - chi
    eq = m == s
    cnteq = jnp.sum(eq.astype(jnp.int32))

    r_iota = jax.lax.broadcasted_iota(jnp.int32, (8, 16384), 0)
    c_iota = jax.lax.broadcasted_iota(jnp.int32, (8, 16384), 1)
    lin = r_iota * 16384 + c_iota

    def tie(_):
        def tb(_, c):
            lo_j, hi_j = c
            mid = (lo_j + hi_j) >> 1
            cnt = jnp.sum((eq & (lin <= mid)).astype(jnp.int32))
            ge = cnt >= need
            return (jnp.where(ge, lo_j, mid), jnp.where(ge, mid, hi_j))
        lo_j, hi_j = jax.lax.fori_loop(
            0, 17, tb, (jnp.int32(-1), jnp.int32(N1 - 1)))
        return hi_j

    jstar = jax.lax.cond(cnteq == need, lambda _: jnp.int32(N1 - 1), tie, 0)
    mask = (m > s) | (eq & (lin <= jstar))
    o_ref[...] = jnp.where(mask, h, 0.0)


def _kw1(h8):
    return pl.pallas_call(
        _kw1_kernel,
        out_shape=jax.ShapeDtypeStruct((8, 16384), jnp.float32),
    )(h8)


def _fc2_kernel(hm_ref, w2_ref, b2_ref, o_ref):
    g = jax.lax.dot_general(
        hm_ref[...], w2_ref[...],
        dimension_numbers=(((1,), (1,)), ((), ())),
        preferred_element_type=jnp.float32) + b2_ref[...]
    m = _mono(g)
    BR = g.shape[0]

    def body(_, c):
        lo, hi, chi = c
        mid = _avg_floor(lo, hi)
        cnt = jnp.sum((m > mid).astype(jnp.int32), axis=1, keepdims=True)
        ge = cnt >= KW2
        return (jnp.where(ge, mid, lo),
                jnp.where(ge, hi, mid),
                jnp.where(ge, chi, cnt))

    lo0 = jnp.full((BR, 1), _I32_MIN, jnp.int32)
    hi0 = jnp.full((BR, 1), _I32_MAX, jnp.int32)
    chi0 = jnp.zeros((BR, 1), jnp.int32)
    lo, hi, chi = jax.lax.fori_loop(0, 32, body, (lo0, hi0, chi0))
    s = hi
    need = KW2 - chi
    eq = m == s
    cnteq = jnp.sum(eq.astype(jnp.int32), axis=1, keepdims=True)
    col = jax.lax.broadcasted_iota(jnp.int32, (BR, C2), 1)

    def tie(_):
        def tb(_, c):
            lo_j, hi_j = c
            mid = (lo_j + hi_j) >> 1
            cnt = jnp.sum((eq & (col <= mid)).astype(jnp.int32),
                          axis=1, keepdims=True)
            ge = cnt >= need
            return (jnp.where(ge, lo_j, mid), jnp.where(ge, mid, hi_j))
        lo_j, hi_j = jax.lax.fori_loop(
            0, 12, tb,
            (jnp.full((BR, 1), -1, jnp.int32),
             jnp.full((BR, 1), C2 - 1, jnp.int32)))
        return hi_j

    jstar = jax.lax.cond(jnp.all(cnteq == need),
                         lambda _: jnp.full((BR, 1), C2 - 1, jnp.int32),
                         tie, 0)
    mask = (m > s) | (eq & (col <= jstar))
    o_ref[...] = jnp.where(mask, g, 0.0)


def _fc2(hm2d, W2, b2w):
    BR = 256
    grid = RW // BR
    return pl.pallas_call(
        _fc2_kernel,
        grid=(grid,),
        in_specs=[
            pl.BlockSpec((BR, Z), lambda i: (i, 0)),
            pl.BlockSpec((C2, Z), lambda i: (0, 0)),
            pl.BlockSpec((1, C2), lambda i: (0, 0)),
        ],
        out_specs=pl.BlockSpec((BR, C2), lambda i: (i, 0)),
        out_shape=jax.ShapeDtypeStruct((RW, C2), jnp.float32),
    )(hm2d, W2, b2w)


def kernel(x, W1, b1, W2, b2):
    x2 = x.reshape(1, Z)
    b1w = b1.reshape(1, N1)
    b2w = b2.reshape(1, C2)
    h = _fc1(x2, W1, b1w)
    hm = _kw1(h.reshape(8, 16384))
    y = _fc2(hm.reshape(RW, Z), W2, b2w)
    return y.reshape(C2, RW)
